# Initial kernel scaffold; baseline (speedup 1.0000x reference)
import jax
import jax.numpy as jnp
from jax.experimental import pallas as pl


def kernel(xyz_bcn, params):
    raise NotImplementedError("write your pallas kernel here")


# speedup vs baseline: 12.4886x; 12.4886x over previous
import functools

import jax
import jax.numpy as jnp
from jax import lax
from jax.experimental import pallas as pl
from jax.experimental.pallas import tpu as pltpu
from jax.experimental.pallas import tpu_sc as plsc

DIM = 128
K = 16
G = 32
CPG = DIM // G
B = 2
N = 4096
NK = N * K
EPS = 1e-5

_KNN_BLK = 256
_NT = 256


def _gn_apply(z, scsh, r0):
    m = scsh[r0:r0 + 1, :]
    v = scsh[r0 + 1:r0 + 2, :]
    gw = scsh[r0 + 2:r0 + 3, :]
    gb = scsh[r0 + 3:r0 + 4, :]
    xn = (z - m) / jnp.sqrt(v + EPS)
    return jax.nn.relu(xn * gw + gb)


def _group_mix(ch, cpg):
    r = lax.broadcasted_iota(jnp.int32, (ch, ch), 0) // cpg
    c = lax.broadcasted_iota(jnp.int32, (ch, ch), 1) // cpg
    return (r == c).astype(jnp.float32)


def _gn_relu_full(z, gw, gb, cpg):
    ch = z.shape[1]
    m = _group_mix(ch, cpg)
    cnt = cpg * z.shape[0]
    mc = jnp.dot(jnp.sum(z, axis=0)[None, :], m,
                 preferred_element_type=jnp.float32,
                 precision=lax.Precision.HIGHEST) / cnt
    d = z - mc
    vc = jnp.dot(jnp.sum(d * d, axis=0)[None, :], m,
                 preferred_element_type=jnp.float32,
                 precision=lax.Precision.HIGHEST) / cnt
    xn = d / jnp.sqrt(vc + EPS)
    return jax.nn.relu(xn * gw[None, :] + gb[None, :])


def _scale_shift(acc, cnt_per_ch):
    ch = acc.shape[2]
    s = acc[:, 0, :].reshape(B, G, ch // G)
    sq = acc[:, 1, :].reshape(B, G, ch // G)
    denom = cnt_per_ch * (ch // G)
    mg = jnp.sum(s, axis=2) / denom
    vg = jnp.sum(sq, axis=2) / denom - mg * mg
    mc = jnp.repeat(mg, ch // G, axis=1)
    vc = jnp.repeat(vg, ch // G, axis=1)
    return mc, vc


def _pack4(m, v, gw, gb):
    gww = jnp.broadcast_to(gw[None], (B, DIM))
    gbb = jnp.broadcast_to(gb[None], (B, DIM))
    z = jnp.zeros((B, 4, DIM), jnp.float32)
    return jnp.concatenate(
        [jnp.stack([m, v, gww, gbb], axis=1), z], axis=1)


def _pack8(m1, v1, g1, b1, m2, v2, g2, b2):
    g1w = jnp.broadcast_to(g1[None], (B, DIM))
    g1b = jnp.broadcast_to(b1[None], (B, DIM))
    g2w = jnp.broadcast_to(g2[None], (B, DIM))
    g2b = jnp.broadcast_to(b2[None], (B, DIM))
    return jnp.stack([m1, v1, g1w, g1b, m2, v2, g2w, g2b], axis=1)


def _acc_stats(acc_ref, z, t):
    s = jnp.sum(z, axis=0)[None, :]
    sq = jnp.sum(z * z, axis=0)[None, :]
    upd = jnp.concatenate([s, sq, jnp.zeros((6, DIM), jnp.float32)], axis=0)

    @pl.when(t == 0)
    def _():
        acc_ref[0] = jnp.zeros((8, DIM), jnp.float32)

    acc_ref[0] += upd


def _embed_body(xyz_ref, wt_ref, b_ref, gw_ref, gb_ref, o_ref):
    z = jnp.dot(xyz_ref[0], wt_ref[...],
                preferred_element_type=jnp.float32) + b_ref[0][None, :]
    o_ref[0] = _gn_relu_full(z, gw_ref[0], gb_ref[0], CPG)


def _embed(xyz, pin):
    return pl.pallas_call(
        _embed_body,
        grid=(B,),
        in_specs=[
            pl.BlockSpec((1, N, 3), lambda b: (b, 0, 0)),
            pl.BlockSpec((3, DIM), lambda b: (0, 0)),
            pl.BlockSpec((1, DIM), lambda b: (0, 0)),
            pl.BlockSpec((1, DIM), lambda b: (0, 0)),
            pl.BlockSpec((1, DIM), lambda b: (0, 0)),
        ],
        out_specs=pl.BlockSpec((1, N, DIM), lambda b: (b, 0, 0)),
        out_shape=jax.ShapeDtypeStruct((B, N, DIM), jnp.float32),
    )(xyz, pin['W'].T, pin['b'][None], pin['gw'][None], pin['gb'][None])


def _qkv_body(y_ref, scsh_ref, wt_ref, bias_ref, xn_ref, qkv_ref):
    xn = _gn_apply(y_ref[0], scsh_ref[0], 0)
    xn_ref[0] = xn
    qkv_ref[0] = jnp.dot(xn, wt_ref[...],
                         preferred_element_type=jnp.float32) + bias_ref[0][None, :]


def _qkv(y, scsh, wt, bias):
    return pl.pallas_call(
        _qkv_body,
        grid=(B,),
        in_specs=[
            pl.BlockSpec((1, N, DIM), lambda b: (b, 0, 0)),
            pl.BlockSpec((1, 8, DIM), lambda b: (b, 0, 0)),
            pl.BlockSpec((DIM, 3 * DIM), lambda b: (0, 0)),
            pl.BlockSpec((1, 3 * DIM), lambda b: (0, 0)),
        ],
        out_specs=[
            pl.BlockSpec((1, N, DIM), lambda b: (b, 0, 0)),
            pl.BlockSpec((1, N, 3 * DIM), lambda b: (b, 0, 0)),
        ],
        out_shape=[
            jax.ShapeDtypeStruct((B, N, DIM), jnp.float32),
            jax.ShapeDtypeStruct((B, N, 3 * DIM), jnp.float32),
        ],
    )(y, scsh, wt, bias)


def _knn_body(fb_ref, f_ref, idx_ref):
    fb = fb_ref[0]
    f = f_ref[0]
    xxb = jnp.sum(fb * fb, axis=1)
    xx = jnp.sum(f * f, axis=1)
    d = xxb[:, None] + xx[None, :] - 2.0 * lax.dot_general(
        fb, f, (((1,), (1,)), ((), ())), preferred_element_type=jnp.float32)
    d = jnp.maximum(d, 0.0)
    iota = lax.broadcasted_iota(jnp.int32, d.shape, 1)
    cols = []
    for _ in range(K):
        m = jnp.min(d, axis=1, keepdims=True)
        am = jnp.min(jnp.where(d == m, iota, N), axis=1)
        cols.append(am)
        d = jnp.where(iota == am[:, None], jnp.float32(3.0e38), d)
    idx_ref[0] = jnp.stack(cols, axis=1)


def _knn(feat):
    c = feat.shape[2]
    return pl.pallas_call(
        _knn_body,
        grid=(B, N // _KNN_BLK),
        in_specs=[
            pl.BlockSpec((1, _KNN_BLK, c), lambda b, t: (b, t, 0)),
            pl.BlockSpec((1, N, c), lambda b, t: (b, 0, 0)),
        ],
        out_specs=pl.BlockSpec((1, _KNN_BLK, K), lambda b, t: (b, t, 0)),
        out_shape=jax.ShapeDtypeStruct((B, N, K), jnp.int32),
    )(feat, feat)


def _sc_gather(table, idx_flat, d):
    m_total = idx_flat.shape[0]
    nw = 32
    mw = m_total // nw
    ch = min(mw, 32768 // d)
    nch = mw // ch
    mesh = plsc.VectorSubcoreMesh(core_axis_name="c", subcore_axis_name="s")

    @functools.partial(
        pl.kernel,
        out_type=jax.ShapeDtypeStruct((m_total, d), jnp.float32),
        mesh=mesh,
        compiler_params=pltpu.CompilerParams(use_tc_tiling_on_sc=False),
        scratch_types=[
            pltpu.VMEM((mw,), jnp.int32),
            pltpu.VMEM((ch, d), jnp.float32),
            pltpu.SemaphoreType.DMA,
        ],
    )
    def gk(table_hbm, idx_hbm, out_hbm, idx_v, rows_v, sem):
        wid = lax.axis_index("s") * 2 + lax.axis_index("c")
        base = wid * mw
        pltpu.sync_copy(idx_hbm.at[pl.ds(base, mw)], idx_v)

        def body(i, carry):
            pltpu.async_copy(
                table_hbm.at[idx_v.at[pl.ds(i * ch, ch)]], rows_v, sem).wait()
            pltpu.sync_copy(rows_v, out_hbm.at[pl.ds(base + i * ch, ch)])
            return carry

        lax.fori_loop(0, nch, body, 0)

    return gk(table, idx_flat)


def _pos_stats_body(xyzg_ref, xyz_ref, poswt_ref, posb_ref, acc_ref):
    dj = xyzg_ref[0][..., :3]
    di = xyz_ref[0][:, None, :]
    delta = (dj - di).reshape(_NT * K, 3)
    pre = jnp.dot(delta, poswt_ref[...],
                  preferred_element_type=jnp.float32) + posb_ref[0][None, :]
    _acc_stats(acc_ref, pre, pl.program_id(1))


def _pos_stats(xyzg, xyz, lp):
    return pl.pallas_call(
        _pos_stats_body,
        grid=(B, N // _NT),
        in_specs=[
            pl.BlockSpec((1, _NT, K, 16), lambda b, t: (b, t, 0, 0)),
            pl.BlockSpec((1, _NT, 3), lambda b, t: (b, t, 0)),
            pl.BlockSpec((3, DIM), lambda b, t: (0, 0)),
            pl.BlockSpec((1, DIM), lambda b, t: (0, 0)),
        ],
        out_specs=pl.BlockSpec((1, 8, DIM), lambda b, t: (b, 0, 0)),
        out_shape=jax.ShapeDtypeStruct((B, 8, DIM), jnp.float32),
    )(xyzg, xyz, lp['posW'].T, lp['posb'][None])


def _edge_common(kv_ref, xyzg_ref, xyz_ref, q_ref, scsh, poswt, posb,
                 attwt, attb):
    tk = _NT * K
    kj = kv_ref[0][..., :DIM].reshape(tk, DIM)
    dj = xyzg_ref[0][..., :3]
    di = xyz_ref[0][:, None, :]
    delta = (dj - di).reshape(tk, 3)
    pre_pos = jnp.dot(delta, poswt, preferred_element_type=jnp.float32) + posb
    pos = _gn_apply(pre_pos, scsh, 0)
    q = q_ref[0]
    qb = jnp.broadcast_to(q[:, None, :], (_NT, K, DIM)).reshape(tk, DIM)
    att_in = kj - qb + pos
    pre_att = jnp.dot(att_in, attwt, preferred_element_type=jnp.float32) + attb
    return pos, pre_att


def _att_stats_body(kv_ref, xyzg_ref, xyz_ref, q_ref, scsh_ref, poswt_ref,
                    posb_ref, attwt_ref, attb_ref, acc_ref):
    _, pre_att = _edge_common(kv_ref, xyzg_ref, xyz_ref, q_ref, scsh_ref[0],
                              poswt_ref[...], posb_ref[0][None, :],
                              attwt_ref[...], attb_ref[0][None, :])
    _acc_stats(acc_ref, pre_att, pl.program_id(1))


def _att_stats(kvg, xyzg, xyz, q, scsh, lp):
    return pl.pallas_call(
        _att_stats_body,
        grid=(B, N // _NT),
        in_specs=[
            pl.BlockSpec((1, _NT, K, 2 * DIM), lambda b, t: (b, t, 0, 0)),
            pl.BlockSpec((1, _NT, K, 16), lambda b, t: (b, t, 0, 0)),
            pl.BlockSpec((1, _NT, 3), lambda b, t: (b, t, 0)),
            pl.BlockSpec((1, _NT, DIM), lambda b, t: (b, t, 0)),
            pl.BlockSpec((1, 8, DIM), lambda b, t: (b, 0, 0)),
            pl.BlockSpec((3, DIM), lambda b, t: (0, 0)),
            pl.BlockSpec((1, DIM), lambda b, t: (0, 0)),
            pl.BlockSpec((DIM, DIM), lambda b, t: (0, 0)),
            pl.BlockSpec((1, DIM), lambda b, t: (0, 0)),
        ],
        out_specs=pl.BlockSpec((1, 8, DIM), lambda b, t: (b, 0, 0)),
        out_shape=jax.ShapeDtypeStruct((B, 8, DIM), jnp.float32),
    )(kvg, xyzg, xyz, q, scsh, lp['posW'].T, lp['posb'][None],
      lp['attW'].T, lp['attb'][None])


def _att_agg_body(kv_ref, xyzg_ref, xyz_ref, q_ref, x_ref, scsh_ref,
                  poswt_ref, posb_ref, attwt_ref, attb_ref, y_ref, acc_ref):
    scsh = scsh_ref[0]
    pos, pre_att = _edge_common(kv_ref, xyzg_ref, xyz_ref, q_ref, scsh,
                                poswt_ref[...], posb_ref[0][None, :],
                                attwt_ref[...], attb_ref[0][None, :])
    a = _gn_apply(pre_att, scsh, 4).reshape(_NT, K, DIM)
    s_att = jnp.sum(a, axis=2)
    mx = jnp.max(s_att, axis=1, keepdims=True)
    e = jnp.exp(s_att - mx)
    w = e / jnp.sum(e, axis=1, keepdims=True)
    vj = kv_ref[0][..., DIM:]
    agg = jnp.sum((vj + pos.reshape(_NT, K, DIM)) * w[:, :, None], axis=1)
    y = x_ref[0] + agg
    y_ref[0] = y
    _acc_stats(acc_ref, y, pl.program_id(1))


def _att_agg(kvg, xyzg, xyz, q, x, scsh, lp):
    return pl.pallas_call(
        _att_agg_body,
        grid=(B, N // _NT),
        in_specs=[
            pl.BlockSpec((1, _NT, K, 2 * DIM), lambda b, t: (b, t, 0, 0)),
            pl.BlockSpec((1, _NT, K, 16), lambda b, t: (b, t, 0, 0)),
            pl.BlockSpec((1, _NT, 3), lambda b, t: (b, t, 0)),
            pl.BlockSpec((1, _NT, DIM), lambda b, t: (b, t, 0)),
            pl.BlockSpec((1, _NT, DIM), lambda b, t: (b, t, 0)),
            pl.BlockSpec((1, 8, DIM), lambda b, t: (b, 0, 0)),
            pl.BlockSpec((3, DIM), lambda b, t: (0, 0)),
            pl.BlockSpec((1, DIM), lambda b, t: (0, 0)),
            pl.BlockSpec((DIM, DIM), lambda b, t: (0, 0)),
            pl.BlockSpec((1, DIM), lambda b, t: (0, 0)),
        ],
        out_specs=[
            pl.BlockSpec((1, _NT, DIM), lambda b, t: (b, t, 0)),
            pl.BlockSpec((1, 8, DIM), lambda b, t: (b, 0, 0)),
        ],
        out_shape=[
            jax.ShapeDtypeStruct((B, N, DIM), jnp.float32),
            jax.ShapeDtypeStruct((B, 8, DIM), jnp.float32),
        ],
    )(kvg, xyzg, xyz, q, x, scsh, lp['posW'].T, lp['posb'][None],
      lp['attW'].T, lp['attb'][None])


def _norm_body(y_ref, scsh_ref, o_ref):
    o_ref[0] = _gn_apply(y_ref[0], scsh_ref[0], 0)


def _norm(y, scsh):
    return pl.pallas_call(
        _norm_body,
        grid=(B,),
        in_specs=[
            pl.BlockSpec((1, N, DIM), lambda b: (b, 0, 0)),
            pl.BlockSpec((1, 8, DIM), lambda b: (b, 0, 0)),
        ],
        out_specs=pl.BlockSpec((1, N, DIM), lambda b: (b, 0, 0)),
        out_shape=jax.ShapeDtypeStruct((B, N, DIM), jnp.float32),
    )(y, scsh)


def _edge_pre1(fg_ref, f_ref, t1wt_ref, t1b_ref):
    tk = _NT * K
    xi = f_ref[0]
    xib = jnp.broadcast_to(xi[:, None, :], (_NT, K, DIM)).reshape(tk, DIM)
    xj = fg_ref[0].reshape(tk, DIM)
    edge = jnp.concatenate([xib, xj - xib], axis=1)
    return jnp.dot(edge, t1wt_ref[...],
                   preferred_element_type=jnp.float32) + t1b_ref[0][None, :]


def _edge1_body(fg_ref, f_ref, t1wt_ref, t1b_ref, acc_ref):
    pre1 = _edge_pre1(fg_ref, f_ref, t1wt_ref, t1b_ref)
    _acc_stats(acc_ref, pre1, pl.program_id(1))


def _edge2_body(fg_ref, f_ref, t1wt_ref, t1b_ref, scsh_ref, t2wt_ref,
                t2b_ref, acc_ref):
    pre1 = _edge_pre1(fg_ref, f_ref, t1wt_ref, t1b_ref)
    y1 = _gn_apply(pre1, scsh_ref[0], 0)
    pre2 = jnp.dot(y1, t2wt_ref[...],
                   preferred_element_type=jnp.float32) + t2b_ref[0][None, :]
    _acc_stats(acc_ref, pre2, pl.program_id(1))


def _edge3_body(fg_ref, f_ref, t1wt_ref, t1b_ref, scsh_ref, t2wt_ref,
                t2b_ref, o_ref):
    pre1 = _edge_pre1(fg_ref, f_ref, t1wt_ref, t1b_ref)
    scsh = scsh_ref[0]
    y1 = _gn_apply(pre1, scsh, 0)
    pre2 = jnp.dot(y1, t2wt_ref[...],
                   preferred_element_type=jnp.float32) + t2b_ref[0][None, :]
    y2 = _gn_apply(pre2, scsh, 4)
    o_ref[0] = jnp.max(y2.reshape(_NT, K, DIM), axis=1) + f_ref[0]


def _edge_specs(n_extra):
    specs = [
        pl.BlockSpec((1, _NT, K, DIM), lambda b, t: (b, t, 0, 0)),
        pl.BlockSpec((1, _NT, DIM), lambda b, t: (b, t, 0)),
        pl.BlockSpec((2 * DIM, DIM), lambda b, t: (0, 0)),
        pl.BlockSpec((1, DIM), lambda b, t: (0, 0)),
    ]
    if n_extra:
        specs += [
            pl.BlockSpec((1, 8, DIM), lambda b, t: (b, 0, 0)),
            pl.BlockSpec((DIM, DIM), lambda b, t: (0, 0)),
            pl.BlockSpec((1, DIM), lambda b, t: (0, 0)),
        ]
    return specs


def _edge1(fg, f, bl):
    return pl.pallas_call(
        _edge1_body,
        grid=(B, N // _NT),
        in_specs=_edge_specs(0),
        out_specs=pl.BlockSpec((1, 8, DIM), lambda b, t: (b, 0, 0)),
        out_shape=jax.ShapeDtypeStruct((B, 8, DIM), jnp.float32),
    )(fg, f, bl['t1W'].T, bl['t1b'][None])


def _edge2(fg, f, scsh, bl):
    return pl.pallas_call(
        _edge2_body,
        grid=(B, N // _NT),
        in_specs=_edge_specs(1),
        out_specs=pl.BlockSpec((1, 8, DIM), lambda b, t: (b, 0, 0)),
        out_shape=jax.ShapeDtypeStruct((B, 8, DIM), jnp.float32),
    )(fg, f, bl['t1W'].T, bl['t1b'][None], scsh, bl['t2W'].T, bl['t2b'][None])


def _edge3(fg, f, scsh, bl):
    return pl.pallas_call(
        _edge3_body,
        grid=(B, N // _NT),
        in_specs=_edge_specs(1),
        out_specs=pl.BlockSpec((1, _NT, DIM), lambda b, t: (b, t, 0)),
        out_shape=jax.ShapeDtypeStruct((B, N, DIM), jnp.float32),
    )(fg, f, bl['t1W'].T, bl['t1b'][None], scsh, bl['t2W'].T, bl['t2b'][None])


def _final_body(f_ref, x_ref, fwt_ref, fb_ref, fgw_ref, fgb_ref,
                hp_ref, hu_ref, o_ref):
    f2 = _gn_relu_full(
        jnp.dot(f_ref[0], fwt_ref[...], preferred_element_type=jnp.float32)
        + fb_ref[0][None, :], fgw_ref[0], fgb_ref[0], CPG)
    x2 = x_ref[0] + f2

    def run_head(h_ref):
        w1t = h_ref[0:DIM, 0:DIM]
        b1 = h_ref[DIM, 0:DIM][None, :]
        g1w = h_ref[DIM + 1, 0:DIM]
        g1b = h_ref[DIM + 2, 0:DIM]
        w2t = h_ref[0:DIM, DIM:DIM + 64]
        b2 = h_ref[DIM, DIM:DIM + 64][None, :]
        g2w = h_ref[DIM + 1, DIM:DIM + 64]
        g2b = h_ref[DIM + 2, DIM:DIM + 64]
        w3t = h_ref[0:64, DIM + 64:DIM + 64 + 8]
        b3 = h_ref[DIM, DIM + 64:DIM + 64 + 8][None, :]
        t1 = _gn_relu_full(jnp.dot(x2, w1t, preferred_element_type=jnp.float32)
                           + b1, g1w, g1b, CPG)
        t2 = _gn_relu_full(jnp.dot(t1, w2t, preferred_element_type=jnp.float32)
                           + b2, g2w, g2b, 2)
        return jnp.dot(t2, w3t, preferred_element_type=jnp.float32) + b3

    hp = run_head(hp_ref[...])
    hu = run_head(hu_ref[...])
    o_ref[0] = jnp.concatenate([hp[:, 0:1], hu[:, 0:3]], axis=1)


def _pack_head(h):
    m = jnp.zeros((DIM + 3, DIM + 64 + 8), jnp.float32)
    m = m.at[0:DIM, 0:DIM].set(h['W1'].T)
    m = m.at[DIM, 0:DIM].set(h['b1'])
    m = m.at[DIM + 1, 0:DIM].set(h['g1w'])
    m = m.at[DIM + 2, 0:DIM].set(h['g1b'])
    m = m.at[0:DIM, DIM:DIM + 64].set(h['W2'].T)
    m = m.at[DIM, DIM:DIM + 64].set(h['b2'])
    m = m.at[DIM + 1, DIM:DIM + 64].set(h['g2w'])
    m = m.at[DIM + 2, DIM:DIM + 64].set(h['g2b'])
    od = h['W3'].shape[0]
    m = m.at[0:64, DIM + 64:DIM + 64 + od].set(h['W3'].T)
    m = m.at[DIM, DIM + 64:DIM + 64 + od].set(h['b3'])
    return m


def _final_heads(f, x, pref, hp, hu):
    hw = DIM + 64 + 8
    return pl.pallas_call(
        _final_body,
        grid=(B,),
        in_specs=[
            pl.BlockSpec((1, N, DIM), lambda b: (b, 0, 0)),
            pl.BlockSpec((1, N, DIM), lambda b: (b, 0, 0)),
            pl.BlockSpec((DIM, DIM), lambda b: (0, 0)),
            pl.BlockSpec((1, DIM), lambda b: (0, 0)),
            pl.BlockSpec((1, DIM), lambda b: (0, 0)),
            pl.BlockSpec((1, DIM), lambda b: (0, 0)),
            pl.BlockSpec((DIM + 3, hw), lambda b: (0, 0)),
            pl.BlockSpec((DIM + 3, hw), lambda b: (0, 0)),
        ],
        out_specs=pl.BlockSpec((1, N, 4), lambda b: (b, 0, 0)),
        out_shape=jax.ShapeDtypeStruct((B, N, 4), jnp.float32),
    )(f, x, pref['fW'].T, pref['fb'][None], pref['fgw'][None],
      pref['fgb'][None], _pack_head(hp), _pack_head(hu))


def kernel(xyz_bcn, params):
    xyz = jnp.transpose(xyz_bcn, (0, 2, 1))
    p = params

    x = _embed(xyz, p['in'])

    idx = _knn(xyz)
    boff = (jnp.arange(B, dtype=jnp.int32) * N)[:, None, None]
    idxf = (idx + boff).reshape(-1)

    xyz_pad = jnp.concatenate(
        [xyz, jnp.zeros((B, N, 13), jnp.float32)], axis=2).reshape(B * N, 16)
    xyzg = _sc_gather(xyz_pad, idxf, 16).reshape(B, N, K, 16)

    y = x
    stats3 = _pack4(jnp.zeros((B, DIM), jnp.float32),
                    jnp.full((B, DIM), 1.0 - EPS, jnp.float32),
                    jnp.ones((DIM,), jnp.float32),
                    jnp.zeros((DIM,), jnp.float32))
    for lp in p['pt']:
        wqkv = jnp.concatenate([lp['qW'], lp['kW'], lp['vW']], axis=0).T
        bqkv = jnp.concatenate([lp['qb'], lp['kb'], lp['vb']])[None]
        xn, qkv = _qkv(y, stats3, wqkv, bqkv)
        q = qkv[..., :DIM]
        kv = qkv[..., DIM:]
        kvg = _sc_gather(kv.reshape(B * N, 2 * DIM), idxf,
                         2 * DIM).reshape(B, N, K, 2 * DIM)
        accp = _pos_stats(xyzg, xyz, lp)
        m1, v1 = _scale_shift(accp, NK)
        scsh1 = _pack4(m1, v1, lp['pos_gw'], lp['pos_gb'])
        acc2 = _att_stats(kvg, xyzg, xyz, q, scsh1, lp)
        m2, v2 = _scale_shift(acc2, NK)
        scsh = _pack8(m1, v1, lp['pos_gw'], lp['pos_gb'],
                      m2, v2, lp['att_gw'], lp['att_gb'])
        y, acc3 = _att_agg(kvg, xyzg, xyz, q, xn, scsh, lp)
        m3, v3 = _scale_shift(acc3, N)
        stats3 = _pack4(m3, v3, lp['nw'], lp['nb'])

    x_final = _norm(y, stats3)

    f = x_final
    for bl in p['ref']['blocks']:
        fidx = _knn(f)
        fidxf = (fidx + boff).reshape(-1)
        fg = _sc_gather(f.reshape(B * N, DIM), fidxf,
                        DIM).reshape(B, N, K, DIM)
        acc1 = _edge1(fg, f, bl)
        m1, v1 = _scale_shift(acc1, NK)
        scsh1 = _pack4(m1, v1, bl['g1w'], bl['g1b'])
        acc2 = _edge2(fg, f, scsh1, bl)
        m2, v2 = _scale_shift(acc2, NK)
        scsh = _pack8(m1, v1, bl['g1w'], bl['g1b'],
                      m2, v2, bl['g2w'], bl['g2b'])
        f = _edge3(fg, f, scsh, bl)

    out4 = _final_heads(f, x_final, p['ref'], p['p_head'], p['uvw_head'])
    return jnp.transpose(out4, (0, 2, 1))
